# Initial kernel scaffold; baseline (speedup 1.0000x reference)
import jax
import jax.numpy as jnp
from jax.experimental import pallas as pl


def kernel(x, coords, W):
    raise NotImplementedError("write your pallas kernel here")


# speedup vs baseline: 8.9169x; 8.9169x over previous
import functools

import jax
import jax.numpy as jnp
from jax.experimental import pallas as pl

B, C_IN, C_OUT, N, K = 4, 32, 32, 4096, 16
NB = 256


def _wx_body(w_ref, x_ref, o_ref):
    o_ref[0] = jnp.dot(w_ref[...], x_ref[0], preferred_element_type=jnp.float32)


def _knn_body(coords_ref, wx_ref, y_ref):
    i = pl.program_id(1)
    ca = coords_ref[0]
    c = coords_ref[0, :, pl.ds(i * NB, NB)]
    inner = jax.lax.dot_general(
        c, ca, (((0,), (0,)), ((), ())),
        preferred_element_type=jnp.float32)
    sq_c = jnp.sum(c * c, axis=0)
    sq_p = jnp.sum(ca * ca, axis=0)
    d = sq_c[:, None] - 2.0 * inner + sq_p[None, :]

    iota = jax.lax.broadcasted_iota(jnp.int32, (NB, N), 1)
    inf = jnp.float32(jnp.inf)
    avail = d
    for _ in range(K):
        m = jnp.min(avail, axis=1, keepdims=True)
        first = jnp.min(jnp.where(avail == m, iota, N), axis=1, keepdims=True)
        avail = jnp.where(iota == first, inf, avail)
    selected = avail == inf

    wx = wx_ref[0]
    outs = []
    for o in range(C_OUT):
        masked = jnp.where(selected, wx[o][None, :], -inf)
        outs.append(jnp.max(masked, axis=1))
    y_ref[0] = jnp.stack(outs, axis=0)


@jax.jit
def kernel(x, coords, W):
    wx = pl.pallas_call(
        _wx_body,
        grid=(B,),
        in_specs=[
            pl.BlockSpec((C_OUT, C_IN), lambda b: (0, 0)),
            pl.BlockSpec((1, C_IN, N), lambda b: (b, 0, 0)),
        ],
        out_specs=pl.BlockSpec((1, C_OUT, N), lambda b: (b, 0, 0)),
        out_shape=jax.ShapeDtypeStruct((B, C_OUT, N), jnp.float32),
    )(W, x)

    y = pl.pallas_call(
        _knn_body,
        grid=(B, N // NB),
        in_specs=[
            pl.BlockSpec((1, 3, N), lambda b, i: (b, 0, 0)),
            pl.BlockSpec((1, C_OUT, N), lambda b, i: (b, 0, 0)),
        ],
        out_specs=pl.BlockSpec((1, C_OUT, NB), lambda b, i: (b, 0, i)),
        out_shape=jax.ShapeDtypeStruct((B, C_OUT, N), jnp.float32),
    )(coords, wx)
    return (y, coords)


# speedup vs baseline: 13.7817x; 1.5456x over previous
import jax
import jax.numpy as jnp
from jax.experimental import pallas as pl

B, C_IN, C_OUT, N, K = 4, 32, 32, 4096, 16
NB = 256


def _wx_body(w_ref, x_ref, o_ref):
    o_ref[0] = jax.lax.dot_general(
        x_ref[0], w_ref[...], (((0,), (1,)), ((), ())),
        preferred_element_type=jnp.float32)


def _knn_body(coords_ref, wxt_ref, y_ref):
    i = pl.program_id(1)
    ca = coords_ref[0]
    c = coords_ref[0, :, pl.ds(i * NB, NB)]
    inner = jax.lax.dot_general(
        c, ca, (((0,), (0,)), ((), ())),
        preferred_element_type=jnp.float32)
    sq_c = jnp.sum(c * c, axis=0)
    sq_p = jnp.sum(ca * ca, axis=0)
    d = sq_c[:, None] - 2.0 * inner + sq_p[None, :]

    iota = jax.lax.broadcasted_iota(jnp.int32, (NB, N), 1)
    inf = jnp.float32(jnp.inf)
    wxt = wxt_ref[0]
    avail = d
    y = None
    for _ in range(K):
        m = jnp.min(avail, axis=1, keepdims=True)
        first = jnp.min(jnp.where(avail == m, iota, N), axis=1, keepdims=True)
        sel = iota == first
        avail = jnp.where(sel, inf, avail)
        onehot = jnp.where(sel, 1.0, 0.0)
        g = jax.lax.dot_general(
            onehot, wxt, (((1,), (0,)), ((), ())),
            preferred_element_type=jnp.float32)
        y = g if y is None else jnp.maximum(y, g)
    y_ref[0] = y


@jax.jit
def kernel(x, coords, W):
    wxt = pl.pallas_call(
        _wx_body,
        grid=(B,),
        in_specs=[
            pl.BlockSpec((C_OUT, C_IN), lambda b: (0, 0)),
            pl.BlockSpec((1, C_IN, N), lambda b: (b, 0, 0)),
        ],
        out_specs=pl.BlockSpec((1, N, C_OUT), lambda b: (b, 0, 0)),
        out_shape=jax.ShapeDtypeStruct((B, N, C_OUT), jnp.float32),
    )(W, x)

    yt = pl.pallas_call(
        _knn_body,
        grid=(B, N // NB),
        in_specs=[
            pl.BlockSpec((1, 3, N), lambda b, i: (b, 0, 0)),
            pl.BlockSpec((1, N, C_OUT), lambda b, i: (b, 0, 0)),
        ],
        out_specs=pl.BlockSpec((1, NB, C_OUT), lambda b, i: (b, i, 0)),
        out_shape=jax.ShapeDtypeStruct((B, N, C_OUT), jnp.float32),
    )(coords, wxt)
    return (yt.transpose(0, 2, 1), coords)


# speedup vs baseline: 14.6558x; 1.0634x over previous
import functools

import jax
import jax.numpy as jnp
from jax import lax
from jax.experimental import pallas as pl
from jax.experimental.pallas import tpu as pltpu
from jax.experimental.pallas import tpu_sc as plsc

B, C_IN, C_OUT, N, K = 4, 32, 32, 4096, 16
NB = 256

NC, NS = 2, 16
NW = NC * NS
PPW = B * N // NW
CH = 8
NCHUNK = PPW // CH


def _wx_body(w_ref, x_ref, o_ref):
    o_ref[0] = jax.lax.dot_general(
        x_ref[0], w_ref[...], (((0,), (1,)), ((), ())),
        preferred_element_type=jnp.float32)


def _knn_body(coords_ref, idx_ref):
    b = pl.program_id(0)
    i = pl.program_id(1)
    ca = coords_ref[0]
    c = coords_ref[0, :, pl.ds(i * NB, NB)]
    inner = jax.lax.dot_general(
        c, ca, (((0,), (0,)), ((), ())),
        preferred_element_type=jnp.float32)
    sq_c = jnp.sum(c * c, axis=0)
    sq_p = jnp.sum(ca * ca, axis=0)
    d = sq_c[:, None] - 2.0 * inner + sq_p[None, :]

    iota = jax.lax.broadcasted_iota(jnp.int32, (NB, N), 1)
    inf = jnp.float32(jnp.inf)
    avail = d
    firsts = []
    for _ in range(K):
        m = jnp.min(avail, axis=1, keepdims=True)
        first = jnp.min(jnp.where(avail == m, iota, N), axis=1, keepdims=True)
        avail = jnp.where(iota == first, inf, avail)
        firsts.append(first)
    idx_ref[0] = jnp.concatenate(firsts, axis=1) + b * N


def _sc_body(table_hbm, idx_hbm, out_hbm, idx_v, rows_v, out_v, sem):
    wid = lax.axis_index("s") * NC + lax.axis_index("c")
    base = wid * PPW
    pltpu.sync_copy(idx_hbm.at[pl.ds(base * K, PPW * K)], idx_v)

    def chunk(g, _):
        pltpu.async_copy(
            table_hbm.at[idx_v.at[pl.ds(g * CH * K, CH * K)]],
            rows_v, sem).wait()
        for p in range(CH):
            a0 = rows_v[p * K, pl.ds(0, 16)]
            a1 = rows_v[p * K, pl.ds(16, 16)]
            for j in range(1, K):
                a0 = jnp.maximum(a0, rows_v[p * K + j, pl.ds(0, 16)])
                a1 = jnp.maximum(a1, rows_v[p * K + j, pl.ds(16, 16)])
            out_v[p, pl.ds(0, 16)] = a0
            out_v[p, pl.ds(16, 16)] = a1
        pltpu.sync_copy(out_v, out_hbm.at[pl.ds(base + g * CH, CH)])
        return ()

    lax.fori_loop(0, NCHUNK, chunk, ())


_sc_gather_max = functools.partial(
    pl.kernel,
    out_type=jax.ShapeDtypeStruct((B * N, C_OUT), jnp.float32),
    mesh=plsc.VectorSubcoreMesh(core_axis_name="c", subcore_axis_name="s"),
    scratch_types=[
        pltpu.VMEM((PPW * K,), jnp.int32),
        pltpu.VMEM((CH * K, C_OUT), jnp.float32),
        pltpu.VMEM((CH, C_OUT), jnp.float32),
        pltpu.SemaphoreType.DMA,
    ],
    compiler_params=pltpu.CompilerParams(use_tc_tiling_on_sc=False),
)(_sc_body)


@jax.jit
def kernel(x, coords, W):
    wxt = pl.pallas_call(
        _wx_body,
        grid=(B,),
        in_specs=[
            pl.BlockSpec((C_OUT, C_IN), lambda b: (0, 0)),
            pl.BlockSpec((1, C_IN, N), lambda b: (b, 0, 0)),
        ],
        out_specs=pl.BlockSpec((1, N, C_OUT), lambda b: (b, 0, 0)),
        out_shape=jax.ShapeDtypeStruct((B, N, C_OUT), jnp.float32),
    )(W, x)

    idx = pl.pallas_call(
        _knn_body,
        grid=(B, N // NB),
        in_specs=[pl.BlockSpec((1, 3, N), lambda b, i: (b, 0, 0))],
        out_specs=pl.BlockSpec((1, NB, K), lambda b, i: (b, i, 0)),
        out_shape=jax.ShapeDtypeStruct((B, N, K), jnp.int32),
    )(coords)

    yt = _sc_gather_max(wxt.reshape(B * N, C_OUT), idx.reshape(B * N * K))
    return (yt.reshape(B, N, C_OUT).transpose(0, 2, 1), coords)


# speedup vs baseline: 15.9892x; 1.0910x over previous
import functools

import jax
import jax.numpy as jnp
from jax import lax
from jax.experimental import pallas as pl
from jax.experimental.pallas import tpu as pltpu
from jax.experimental.pallas import tpu_sc as plsc

B, C_IN, C_OUT, N, K = 4, 32, 32, 4096, 16
NB = 256

NC, NS = 2, 16
NW = NC * NS
PPW = B * N // NW
CH = 8
NCHUNK = PPW // CH


def _wx_body(w_ref, x_ref, o_ref):
    o_ref[0] = jax.lax.dot_general(
        x_ref[0], w_ref[...], (((0,), (1,)), ((), ())),
        preferred_element_type=jnp.float32)


def _knn_body(coords_ref, idx_ref):
    b = pl.program_id(0)
    i = pl.program_id(1)
    ca = coords_ref[0]
    c = coords_ref[0, :, pl.ds(i * NB, NB)]
    inner = jax.lax.dot_general(
        c, ca, (((0,), (0,)), ((), ())),
        preferred_element_type=jnp.float32)
    sq_c = jnp.sum(c * c, axis=0)
    sq_p = jnp.sum(ca * ca, axis=0)
    d = sq_c[:, None] - 2.0 * inner + sq_p[None, :]

    H = N // 2
    inf = jnp.float32(jnp.inf)
    i_l = jax.lax.broadcasted_iota(jnp.int32, (NB, H), 1)
    i_r = i_l + H
    d_l = d[:, :H]
    d_r = d[:, H:]
    cmp = d_l <= d_r
    lo = jnp.minimum(d_l, d_r)
    hi = jnp.maximum(d_l, d_r)
    loidx = jnp.where(cmp, i_l, i_r)
    hiidx = jnp.where(cmp, i_r, i_l)
    big = jnp.int32(1 << 30)
    firsts = []
    for _ in range(K):
        m = jnp.min(lo, axis=1, keepdims=True)
        ft = jnp.min(jnp.where(lo == m, loidx, big), axis=1, keepdims=True)
        sel = loidx == ft
        lo = jnp.where(sel, hi, lo)
        loidx = jnp.where(sel, hiidx, loidx)
        hi = jnp.where(sel, inf, hi)
        firsts.append(ft)
    idx_ref[0] = jnp.concatenate(firsts, axis=1) + b * N


def _sc_body(table_hbm, idx_hbm, out_hbm, idx_v, rows_v, out_v, sem):
    wid = lax.axis_index("s") * NC + lax.axis_index("c")
    base = wid * PPW
    pltpu.sync_copy(idx_hbm.at[pl.ds(base * K, PPW * K)], idx_v)

    def chunk(g, _):
        pltpu.async_copy(
            table_hbm.at[idx_v.at[pl.ds(g * CH * K, CH * K)]],
            rows_v, sem).wait()
        for p in range(CH):
            a0 = rows_v[p * K, pl.ds(0, 16)]
            a1 = rows_v[p * K, pl.ds(16, 16)]
            for j in range(1, K):
                a0 = jnp.maximum(a0, rows_v[p * K + j, pl.ds(0, 16)])
                a1 = jnp.maximum(a1, rows_v[p * K + j, pl.ds(16, 16)])
            out_v[p, pl.ds(0, 16)] = a0
            out_v[p, pl.ds(16, 16)] = a1
        pltpu.sync_copy(out_v, out_hbm.at[pl.ds(base + g * CH, CH)])
        return ()

    lax.fori_loop(0, NCHUNK, chunk, ())


_sc_gather_max = functools.partial(
    pl.kernel,
    out_type=jax.ShapeDtypeStruct((B * N, C_OUT), jnp.float32),
    mesh=plsc.VectorSubcoreMesh(core_axis_name="c", subcore_axis_name="s"),
    scratch_types=[
        pltpu.VMEM((PPW * K,), jnp.int32),
        pltpu.VMEM((CH * K, C_OUT), jnp.float32),
        pltpu.VMEM((CH, C_OUT), jnp.float32),
        pltpu.SemaphoreType.DMA,
    ],
    compiler_params=pltpu.CompilerParams(use_tc_tiling_on_sc=False),
)(_sc_body)


@jax.jit
def kernel(x, coords, W):
    wxt = pl.pallas_call(
        _wx_body,
        grid=(B,),
        in_specs=[
            pl.BlockSpec((C_OUT, C_IN), lambda b: (0, 0)),
            pl.BlockSpec((1, C_IN, N), lambda b: (b, 0, 0)),
        ],
        out_specs=pl.BlockSpec((1, N, C_OUT), lambda b: (b, 0, 0)),
        out_shape=jax.ShapeDtypeStruct((B, N, C_OUT), jnp.float32),
    )(W, x)

    idx = pl.pallas_call(
        _knn_body,
        grid=(B, N // NB),
        in_specs=[pl.BlockSpec((1, 3, N), lambda b, i: (b, 0, 0))],
        out_specs=pl.BlockSpec((1, NB, K), lambda b, i: (b, i, 0)),
        out_shape=jax.ShapeDtypeStruct((B, N, K), jnp.int32),
    )(coords)

    yt = _sc_gather_max(wxt.reshape(B * N, C_OUT), idx.reshape(B * N * K))
    return (yt.reshape(B, N, C_OUT).transpose(0, 2, 1), coords)
